# baseline (speedup 1.0000x reference)
import functools

import jax
import jax.numpy as jnp
from jax import lax
from jax.experimental import pallas as pl
from jax.experimental.pallas import tpu as pltpu

NUM_UNIQUE = 1000000
EMBED_DIM = 64
BATCH = 16384


def _make_tc_gather(n):
    def body(idx_s, table_any, out_any, rows_v, sem, sem_out):
        def fire(j, carry):
            t = idx_s[j]
            pltpu.make_async_copy(
                table_any.at[pl.ds(t, 1), :],
                rows_v.at[pl.ds(j, 1), :],
                sem,
            ).start()
            return carry

        lax.fori_loop(0, n, fire, 0, unroll=4)

        def drain(j, carry):
            pltpu.make_async_copy(
                table_any.at[pl.ds(0, 1), :],
                rows_v.at[pl.ds(0, 1), :],
                sem,
            ).wait()
            return carry

        lax.fori_loop(0, n, drain, 0, unroll=4)

        out_cp = pltpu.make_async_copy(rows_v, out_any, sem_out)
        out_cp.start()
        out_cp.wait()

    return pl.pallas_call(
        body,
        out_shape=jax.ShapeDtypeStruct((n, EMBED_DIM), jnp.float32),
        in_specs=[
            pl.BlockSpec(memory_space=pltpu.SMEM),
            pl.BlockSpec(memory_space=pltpu.HBM),
        ],
        out_specs=pl.BlockSpec(memory_space=pltpu.HBM),
        scratch_shapes=[
            pltpu.VMEM((n, EMBED_DIM), jnp.float32),
            pltpu.SemaphoreType.DMA,
            pltpu.SemaphoreType.DMA,
        ],
    )


_tc_gather = _make_tc_gather(BATCH)


def kernel(x, table):
    return _tc_gather(x.astype(jnp.int32), table)


# speedup vs baseline: 1.0580x; 1.0580x over previous
import functools

import jax
import jax.numpy as jnp
from jax import lax
from jax.experimental import pallas as pl
from jax.experimental.pallas import tpu as pltpu

NUM_UNIQUE = 1000000
EMBED_DIM = 64
BATCH = 16384


def _make_tc_gather(n):
    def body(idx_s, table_any, out_any, rows_v, sem, sem_out):
        def fire(j, carry):
            t = idx_s[j]
            pltpu.make_async_copy(
                table_any.at[pl.ds(t, 1), :],
                rows_v.at[pl.ds(j, 1), :],
                sem,
            ).start()
            return carry

        lax.fori_loop(0, n, fire, 0, unroll=8)

        def drain(j, carry):
            pltpu.make_async_copy(
                table_any.at[pl.ds(0, 64), :],
                rows_v.at[pl.ds(0, 64), :],
                sem,
            ).wait()
            return carry

        lax.fori_loop(0, n // 64, drain, 0, unroll=4)

        out_cp = pltpu.make_async_copy(rows_v, out_any, sem_out)
        out_cp.start()
        out_cp.wait()

    return pl.pallas_call(
        body,
        out_shape=jax.ShapeDtypeStruct((n, EMBED_DIM), jnp.float32),
        in_specs=[
            pl.BlockSpec(memory_space=pltpu.SMEM),
            pl.BlockSpec(memory_space=pltpu.HBM),
        ],
        out_specs=pl.BlockSpec(memory_space=pltpu.HBM),
        scratch_shapes=[
            pltpu.VMEM((n, EMBED_DIM), jnp.float32),
            pltpu.SemaphoreType.DMA,
            pltpu.SemaphoreType.DMA,
        ],
    )


_tc_gather = _make_tc_gather(BATCH)


def kernel(x, table):
    return _tc_gather(x.astype(jnp.int32), table)


# speedup vs baseline: 1.0906x; 1.0307x over previous
import functools

import jax
import jax.numpy as jnp
from jax import lax
from jax.experimental import pallas as pl
from jax.experimental.pallas import tpu as pltpu
from jax.experimental.pallas import tpu_sc as plsc

NUM_UNIQUE = 1000000
EMBED_DIM = 64
BATCH = 16384

_NC = 2
_NS = 16
_NW = _NC * _NS
_N_SC = 8192
_N_TC = BATCH - _N_SC
_B_PER_W = _N_SC // _NW


def _make_sc_gather():
    mesh = plsc.VectorSubcoreMesh(core_axis_name="c", subcore_axis_name="s")

    @functools.partial(
        pl.kernel,
        mesh=mesh,
        out_type=jax.ShapeDtypeStruct((_N_SC, EMBED_DIM), jnp.float32),
        scratch_types=[
            pltpu.VMEM((_B_PER_W // 128, 128), jnp.int32),
            pltpu.VMEM((_B_PER_W // 8, 8, EMBED_DIM), jnp.float32),
            pltpu.SemaphoreType.DMA,
            pltpu.SemaphoreType.DMA,
            pltpu.SemaphoreType.DMA,
            pltpu.SemaphoreType.DMA,
        ],
        compiler_params=pltpu.CompilerParams(disable_bounds_checks=True),
    )
    def k(table_hbm, idx_hbm, out_hbm, idx_v, rows_v, s0, s1, s2, s3):
        wid = lax.axis_index("s") * _NC + lax.axis_index("c")
        sems = (s0, s1, s2, s3)
        pltpu.sync_copy(idx_hbm.at[wid], idx_v)

        def fire(g, carry):
            tv = idx_v[g >> 3, pl.ds((g & 7) * 16, 16)]
            for l in range(16):
                pltpu.async_copy(
                    table_hbm.at[tv[l]],
                    rows_v.at[2 * g + (l // 8), l & 7],
                    sems[l % 4],
                )
            return carry

        lax.fori_loop(0, _B_PER_W // 16, fire, 0)

        for q in range(4):
            pltpu.make_async_copy(
                table_hbm.at[pl.ds(0, _B_PER_W // 32), :],
                rows_v.at[pl.ds(0, _B_PER_W // 32)],
                sems[q],
            ).wait()

        out3 = out_hbm.reshape(_N_SC // 8, 8, EMBED_DIM)
        pltpu.sync_copy(rows_v, out3.at[pl.ds(wid * (_B_PER_W // 8),
                                              _B_PER_W // 8)])

    return k


def _make_tc_gather(n):
    def body(idx_s, table_any, out_any, rows_v, sem, sem_out):
        def fire(j, carry):
            t = idx_s[j]
            pltpu.make_async_copy(
                table_any.at[pl.ds(t, 1), :],
                rows_v.at[pl.ds(j, 1), :],
                sem,
            ).start()
            return carry

        lax.fori_loop(0, n, fire, 0, unroll=8)

        def drain(j, carry):
            pltpu.make_async_copy(
                table_any.at[pl.ds(0, 64), :],
                rows_v.at[pl.ds(0, 64), :],
                sem,
            ).wait()
            return carry

        lax.fori_loop(0, n // 64, drain, 0, unroll=4)

        out_cp = pltpu.make_async_copy(rows_v, out_any, sem_out)
        out_cp.start()
        out_cp.wait()

    return pl.pallas_call(
        body,
        out_shape=jax.ShapeDtypeStruct((n, EMBED_DIM), jnp.float32),
        in_specs=[
            pl.BlockSpec(memory_space=pltpu.SMEM),
            pl.BlockSpec(memory_space=pltpu.HBM),
        ],
        out_specs=pl.BlockSpec(memory_space=pltpu.HBM),
        scratch_shapes=[
            pltpu.VMEM((n, EMBED_DIM), jnp.float32),
            pltpu.SemaphoreType.DMA,
            pltpu.SemaphoreType.DMA,
        ],
    )


_sc_gather = _make_sc_gather()
_tc_gather = _make_tc_gather(_N_TC)


def kernel(x, table):
    xi = x.astype(jnp.int32)
    idx3 = xi[:_N_SC].reshape(_NW, _B_PER_W // 128, 128)
    out_sc = _sc_gather(table, idx3)
    out_tc = _tc_gather(xi[_N_SC:], table)
    return jnp.concatenate([out_sc, out_tc], axis=0)
